# baseline (speedup 1.0000x reference)
import functools

import jax
import jax.numpy as jnp
from jax import lax
from jax.experimental import pallas as pl
from jax.experimental.pallas import tpu as pltpu
from jax.experimental.pallas import tpu_sc as plsc

_B = 16384
_L = 200
_DIM = 32
_DT = _DIM // 8
_BT = _B // 128

_NC = 2
_NS = 16
_NW = _NC * _NS
_TPW = _BT // _NW
_SL = 4
_LC = _L // _SL
_NBLK = _TPW * _LC


def _gather_body(xt_hbm, table_hbm, out_hbm,
                 idx0, idx1, rows0, rows1, tiles0, tiles1,
                 semi0, semi1, semg0, semg1, semw0, semw1):
    wid = lax.axis_index("s") * _NC + lax.axis_index("c")
    idx_v = (idx0, idx1)
    rows_v = (rows0, rows1)
    tiles_v = (tiles0, tiles1)
    semi = (semi0, semi1)
    semg = (semg0, semg1)
    semw = (semw0, semw1)

    iota = lax.iota(jnp.int32, 16)
    rowbase = [iota + 16 * g for g in range(8)]

    def coords(i):
        g = lax.div(i, _LC)
        l0 = _SL * lax.rem(i, _LC)
        return l0, wid * _TPW + g

    def fire_idx(p, i):
        l0, bt = coords(i)
        pltpu.async_copy(xt_hbm.at[pl.ds(l0, _SL), pl.ds(bt * 128, 128)],
                         idx_v[p], semi[p])

    def drain_idx(p):
        pltpu.make_async_copy(xt_hbm.at[pl.ds(0, _SL), pl.ds(0, 128)],
                              idx_v[p], semi[p]).wait()

    def fire_gather(p):
        for sl in range(_SL):
            pltpu.async_copy(table_hbm.at[idx_v[p].at[sl]],
                             rows_v[p].at[sl], semg[p])

    def drain_gather(p):
        for sl in range(_SL):
            pltpu.make_async_copy(table_hbm.at[idx_v[p].at[sl]],
                                  rows_v[p].at[sl], semg[p]).wait()

    def swizzle(p):
        for sl in range(_SL):
            src = rows_v[p].at[sl]
            dst = tiles_v[p].at[sl]

            def d0_body(d0, carry):
                m = d0 + iota
                m = jnp.where(m >= _DIM, m - _DIM, m)
                dtv = lax.shift_right_logical(m, 3)
                inner = lax.shift_left(lax.bitwise_and(m, 7), 7)
                for g in range(8):
                    vals = plsc.load_gather(src, [rowbase[g], m])
                    plsc.store_scatter(dst, [dtv, inner + rowbase[g]],
                                       vals)
                return carry

            lax.fori_loop(0, _DIM, d0_body, 0)

    def fire_writes(p, i):
        l0, bt = coords(i)
        pltpu.async_copy(
            tiles_v[p],
            out_hbm.at[pl.ds(l0, _SL), pl.ds(0, _DT),
                       pl.ds(bt * 1024, 1024)],
            semw[p])

    def drain_writes(p):
        pltpu.make_async_copy(
            tiles_v[p],
            out_hbm.at[pl.ds(0, _SL), pl.ds(0, _DT), pl.ds(0, 1024)],
            semw[p]).wait()

    fire_idx(0, 0)
    drain_idx(0)
    fire_gather(0)
    fire_idx(1, 1)

    def step(s, carry):
        for p in (0, 1):
            i = 2 * s + p

            @pl.when(s >= 1)
            def _(p=p):
                drain_writes(p)

            drain_gather(p)

            @pl.when(i + 1 < _NBLK)
            def _(p=p):
                drain_idx(1 - p)
                fire_gather(1 - p)

            @pl.when(i + 2 < _NBLK)
            def _(p=p, i=i):
                fire_idx(p, i + 2)

            swizzle(p)
            fire_writes(p, i)

        return carry

    lax.fori_loop(0, _NBLK // 2, step, 0)
    drain_writes(0)
    drain_writes(1)


@jax.jit
def _embedding_lookup(xt, table):
    mesh = plsc.VectorSubcoreMesh(core_axis_name="c", subcore_axis_name="s")
    return pl.kernel(
        _gather_body,
        mesh=mesh,
        out_type=jax.ShapeDtypeStruct((_L, _DT, _BT * 1024), jnp.float32),
        scratch_types=[
            pltpu.VMEM((_SL, 128), jnp.int32),
            pltpu.VMEM((_SL, 128), jnp.int32),
            pltpu.VMEM((_SL, 128, _DIM), jnp.float32),
            pltpu.VMEM((_SL, 128, _DIM), jnp.float32),
            pltpu.VMEM((_SL, _DT, 1024), jnp.float32),
            pltpu.VMEM((_SL, _DT, 1024), jnp.float32),
            pltpu.SemaphoreType.DMA,
            pltpu.SemaphoreType.DMA,
            pltpu.SemaphoreType.DMA,
            pltpu.SemaphoreType.DMA,
            pltpu.SemaphoreType.DMA,
            pltpu.SemaphoreType.DMA,
        ],
        compiler_params=pltpu.CompilerParams(use_tc_tiling_on_sc=False,
                                             needs_layout_passes=False),
    )(xt, table)


def kernel(x, table):
    xt = x.T.astype(jnp.int32)
    q = _embedding_lookup(xt, table)
    q = q.reshape(_L, _DT, _BT, 8, 128)
    return q.transpose(2, 4, 0, 1, 3).reshape(_B, _L, _DIM)


# speedup vs baseline: 1.6051x; 1.6051x over previous
import functools

import jax
import jax.numpy as jnp
from jax import lax
from jax.experimental import pallas as pl
from jax.experimental.pallas import tpu as pltpu
from jax.experimental.pallas import tpu_sc as plsc

_B = 16384
_L = 200
_DIM = 32
_DT = _DIM // 8
_BT = _B // 128

_NC = 2
_NS = 16
_NW = _NC * _NS
_TPW = _BT // _NW
_SL = 4
_LC = _L // _SL
_NBLK = _TPW * _LC


def _gather_body(xt_hbm, table_hbm, out_hbm,
                 idx0, idx1, rows0, rows1, tiles0, tiles1,
                 semi0, semi1, semg0, semg1, semw0, semw1):
    wid = lax.axis_index("s") * _NC + lax.axis_index("c")
    idx_v = (idx0, idx1)
    rows_v = (rows0, rows1)
    tiles_v = (tiles0, tiles1)
    semi = (semi0, semi1)
    semg = (semg0, semg1)
    semw = (semw0, semw1)

    iota = lax.iota(jnp.int32, 16)
    rowbase = [iota + 16 * g for g in range(8)]

    def coords(i):
        g = lax.div(i, _LC)
        l0 = _SL * lax.rem(i, _LC)
        return l0, wid * _TPW + g

    def fire_idx(p, i):
        l0, bt = coords(i)
        pltpu.async_copy(xt_hbm.at[pl.ds(l0, _SL), pl.ds(bt * 128, 128)],
                         idx_v[p], semi[p])

    def drain_idx(p):
        pltpu.make_async_copy(xt_hbm.at[pl.ds(0, _SL), pl.ds(0, 128)],
                              idx_v[p], semi[p]).wait()

    def fire_gather(p):
        for sl in range(_SL):
            pltpu.async_copy(table_hbm.at[idx_v[p].at[sl]],
                             rows_v[p].at[sl], semg[p])

    def drain_gather(p):
        for sl in range(_SL):
            pltpu.make_async_copy(table_hbm.at[idx_v[p].at[sl]],
                                  rows_v[p].at[sl], semg[p]).wait()

    def swizzle(p):
        for sl in range(_SL):
            src = rows_v[p].at[sl]
            dst = tiles_v[p].at[sl]

            def d0_body(c, carry):
                for u in range(8):
                    m = 8 * c + u + iota
                    m = jnp.where(m >= _DIM, m - _DIM, m)
                    dtv = lax.shift_right_logical(m, 3)
                    inner = lax.shift_left(lax.bitwise_and(m, 7), 7)
                    vals = [plsc.load_gather(src, [rowbase[g], m])
                            for g in range(8)]
                    for g in range(8):
                        plsc.store_scatter(dst, [dtv, inner + rowbase[g]],
                                           vals[g])
                return carry

            lax.fori_loop(0, _DIM // 8, d0_body, 0)

    def fire_writes(p, i):
        l0, bt = coords(i)
        pltpu.async_copy(
            tiles_v[p],
            out_hbm.at[pl.ds(l0, _SL), pl.ds(0, _DT),
                       pl.ds(bt * 1024, 1024)],
            semw[p])

    def drain_writes(p):
        pltpu.make_async_copy(
            tiles_v[p],
            out_hbm.at[pl.ds(0, _SL), pl.ds(0, _DT), pl.ds(0, 1024)],
            semw[p]).wait()

    fire_idx(0, 0)
    drain_idx(0)
    fire_gather(0)
    fire_idx(1, 1)

    def step(s, carry):
        for p in (0, 1):
            i = 2 * s + p

            @pl.when(s >= 1)
            def _(p=p):
                drain_writes(p)

            drain_gather(p)

            @pl.when(i + 1 < _NBLK)
            def _(p=p):
                drain_idx(1 - p)
                fire_gather(1 - p)

            @pl.when(i + 2 < _NBLK)
            def _(p=p, i=i):
                fire_idx(p, i + 2)

            swizzle(p)
            fire_writes(p, i)

        return carry

    lax.fori_loop(0, _NBLK // 2, step, 0)
    drain_writes(0)
    drain_writes(1)


@jax.jit
def _embedding_lookup(xt, table):
    mesh = plsc.VectorSubcoreMesh(core_axis_name="c", subcore_axis_name="s")
    return pl.kernel(
        _gather_body,
        mesh=mesh,
        out_type=jax.ShapeDtypeStruct((_L, _DT, _BT * 1024), jnp.float32),
        scratch_types=[
            pltpu.VMEM((_SL, 128), jnp.int32),
            pltpu.VMEM((_SL, 128), jnp.int32),
            pltpu.VMEM((_SL, 128, _DIM), jnp.float32),
            pltpu.VMEM((_SL, 128, _DIM), jnp.float32),
            pltpu.VMEM((_SL, _DT, 1024), jnp.float32),
            pltpu.VMEM((_SL, _DT, 1024), jnp.float32),
            pltpu.SemaphoreType.DMA,
            pltpu.SemaphoreType.DMA,
            pltpu.SemaphoreType.DMA,
            pltpu.SemaphoreType.DMA,
            pltpu.SemaphoreType.DMA,
            pltpu.SemaphoreType.DMA,
        ],
        compiler_params=pltpu.CompilerParams(use_tc_tiling_on_sc=False,
                                             needs_layout_passes=False),
    )(xt, table)


def kernel(x, table):
    xt = x.T.astype(jnp.int32)
    q = _embedding_lookup(xt, table)
    q = q.reshape(_L, _DT, _BT, 8, 128)
    return q.transpose(2, 4, 0, 1, 3).reshape(_B, _L, _DIM)


# speedup vs baseline: 1.6160x; 1.0068x over previous
import functools

import jax
import jax.numpy as jnp
from jax import lax
from jax.experimental import pallas as pl
from jax.experimental.pallas import tpu as pltpu
from jax.experimental.pallas import tpu_sc as plsc

_B = 16384
_L = 200
_DIM = 32
_DT = _DIM // 8
_BT = _B // 128

_NC = 2
_NS = 16
_NW = _NC * _NS
_TPW = _BT // _NW
_SL = 4
_LC = _L // _SL
_NBLK = _TPW * _LC


def _gather_body(xt_hbm, table_hbm, out_hbm,
                 idx0, idx1, idx4a, idx4b, rows0, rows1, tiles0, tiles1,
                 semi0, semi1, semg0, semg1, semw0, semw1):
    wid = lax.axis_index("s") * _NC + lax.axis_index("c")
    idx_v = (idx0, idx1)
    idx4_v = (idx4a, idx4b)
    rows_v = (rows0, rows1)
    tiles_v = (tiles0, tiles1)
    semi = (semi0, semi1)
    semg = (semg0, semg1)
    semw = (semw0, semw1)

    iota = lax.iota(jnp.int32, 16)
    rowbase = [iota + 16 * g for g in range(8)]

    def coords(i):
        g = lax.div(i, _LC)
        l0 = _SL * lax.rem(i, _LC)
        return l0, wid * _TPW + g

    def fire_idx(p, i):
        l0, bt = coords(i)
        pltpu.async_copy(xt_hbm.at[pl.ds(l0, _SL), pl.ds(bt * 128, 128)],
                         idx_v[p], semi[p])

    def drain_idx(p):
        pltpu.make_async_copy(xt_hbm.at[pl.ds(0, _SL), pl.ds(0, 128)],
                              idx_v[p], semi[p]).wait()

    def scale_idx(p):
        for sl in range(_SL):
            for c in range(8):
                v = idx_v[p][sl, pl.ds(16 * c, 16)]
                idx4_v[p][sl, pl.ds(16 * c, 16)] = lax.shift_left(v, 2)

    def fire_gather(p):
        for sl in range(_SL):
            pltpu.async_copy(table_hbm.at[idx4_v[p].at[sl]],
                             rows_v[p].at[sl], semg[p])

    def drain_gather(p):
        for sl in range(_SL):
            pltpu.make_async_copy(table_hbm.at[idx4_v[p].at[sl]],
                                  rows_v[p].at[sl], semg[p]).wait()

    def swizzle(p):
        for sl in range(_SL):
            src = rows_v[p].at[sl]
            dst = tiles_v[p].at[sl]

            def d0_body(c, carry):
                for u in range(8):
                    m = 8 * c + u + iota
                    m = jnp.where(m >= _DIM, m - _DIM, m)
                    dtv = lax.shift_right_logical(m, 3)
                    inner = lax.shift_left(lax.bitwise_and(m, 7), 7)
                    vals = [plsc.load_gather(src, [rowbase[g], m])
                            for g in range(8)]
                    for g in range(8):
                        plsc.store_scatter(dst, [dtv, inner + rowbase[g]],
                                           vals[g])
                return carry

            lax.fori_loop(0, _DIM // 8, d0_body, 0)

    def fire_writes(p, i):
        l0, bt = coords(i)
        pltpu.async_copy(
            tiles_v[p],
            out_hbm.at[pl.ds(l0, _SL), pl.ds(0, _DT),
                       pl.ds(bt * 1024, 1024)],
            semw[p])

    def drain_writes(p):
        pltpu.make_async_copy(
            tiles_v[p],
            out_hbm.at[pl.ds(0, _SL), pl.ds(0, _DT), pl.ds(0, 1024)],
            semw[p]).wait()

    fire_idx(0, 0)
    drain_idx(0)
    scale_idx(0)
    fire_gather(0)
    fire_idx(1, 1)

    def step(s, carry):
        for p in (0, 1):
            i = 2 * s + p

            @pl.when(s >= 1)
            def _(p=p):
                drain_writes(p)

            drain_gather(p)

            @pl.when(i + 1 < _NBLK)
            def _(p=p):
                drain_idx(1 - p)
                scale_idx(1 - p)
                fire_gather(1 - p)

            @pl.when(i + 2 < _NBLK)
            def _(p=p, i=i):
                fire_idx(p, i + 2)

            swizzle(p)
            fire_writes(p, i)

        return carry

    lax.fori_loop(0, _NBLK // 2, step, 0)
    drain_writes(0)
    drain_writes(1)


@jax.jit
def _embedding_lookup(xt, table):
    mesh = plsc.VectorSubcoreMesh(core_axis_name="c", subcore_axis_name="s")
    return pl.kernel(
        _gather_body,
        mesh=mesh,
        out_type=jax.ShapeDtypeStruct((_L, _DT, _BT * 1024), jnp.float32),
        scratch_types=[
            pltpu.VMEM((_SL, 128), jnp.int32),
            pltpu.VMEM((_SL, 128), jnp.int32),
            pltpu.VMEM((_SL, 128), jnp.int32),
            pltpu.VMEM((_SL, 128), jnp.int32),
            pltpu.VMEM((_SL, 128, _DIM), jnp.float32),
            pltpu.VMEM((_SL, 128, _DIM), jnp.float32),
            pltpu.VMEM((_SL, _DT, 1024), jnp.float32),
            pltpu.VMEM((_SL, _DT, 1024), jnp.float32),
            pltpu.SemaphoreType.DMA,
            pltpu.SemaphoreType.DMA,
            pltpu.SemaphoreType.DMA,
            pltpu.SemaphoreType.DMA,
            pltpu.SemaphoreType.DMA,
            pltpu.SemaphoreType.DMA,
        ],
        compiler_params=pltpu.CompilerParams(use_tc_tiling_on_sc=False,
                                             needs_layout_passes=False),
    )(xt, table)


def kernel(x, table):
    xt = x.T.astype(jnp.int32)
    tp = jnp.pad(table, ((0, 0), (0, 96))).reshape(4 * 1000000, 32)
    q = _embedding_lookup(xt, tp)
    q = q.reshape(_L, _DT, _BT, 8, 128)
    return q.transpose(2, 4, 0, 1, 3).reshape(_B, _L, _DIM)


# speedup vs baseline: 1.6258x; 1.0060x over previous
import functools

import jax
import jax.numpy as jnp
from jax import lax
from jax.experimental import pallas as pl
from jax.experimental.pallas import tpu as pltpu
from jax.experimental.pallas import tpu_sc as plsc

_B = 16384
_L = 200
_DIM = 32
_DT = _DIM // 8
_BT = _B // 128

_NC = 2
_NS = 16
_NW = _NC * _NS
_TPW = _BT // _NW
_SL = 5
_LC = _L // _SL
_NBLK = _TPW * _LC


def _gather_body(xt_hbm, table_hbm, out_hbm,
                 idx0, idx1, idx4a, idx4b, rows0, rows1, tiles0, tiles1,
                 semi0, semi1, semg0, semg1, semw0, semw1):
    wid = lax.axis_index("s") * _NC + lax.axis_index("c")
    idx_v = (idx0, idx1)
    idx4_v = (idx4a, idx4b)
    rows_v = (rows0, rows1)
    tiles_v = (tiles0, tiles1)
    semi = (semi0, semi1)
    semg = (semg0, semg1)
    semw = (semw0, semw1)

    iota = lax.iota(jnp.int32, 16)
    rowbase = [iota + 16 * g for g in range(8)]

    def coords(i):
        g = lax.div(i, _LC)
        l0 = _SL * lax.rem(i, _LC)
        return l0, wid * _TPW + g

    def fire_idx(p, i):
        l0, bt = coords(i)
        pltpu.async_copy(xt_hbm.at[pl.ds(l0, _SL), pl.ds(bt * 128, 128)],
                         idx_v[p], semi[p])

    def drain_idx(p):
        pltpu.make_async_copy(xt_hbm.at[pl.ds(0, _SL), pl.ds(0, 128)],
                              idx_v[p], semi[p]).wait()

    def scale_idx(p):
        for sl in range(_SL):
            for c in range(8):
                v = idx_v[p][sl, pl.ds(16 * c, 16)]
                idx4_v[p][sl, pl.ds(16 * c, 16)] = lax.shift_left(v, 2)

    def fire_gather(p):
        for sl in range(_SL):
            pltpu.async_copy(table_hbm.at[idx4_v[p].at[sl]],
                             rows_v[p].at[sl], semg[p])

    def drain_gather(p):
        for sl in range(_SL):
            pltpu.make_async_copy(table_hbm.at[idx4_v[p].at[sl]],
                                  rows_v[p].at[sl], semg[p]).wait()

    def swizzle(p):
        for sl in range(_SL):
            src = rows_v[p].at[sl]
            dst = tiles_v[p].at[sl]

            def d0_body(c, carry):
                for u in range(8):
                    m = 8 * c + u + iota
                    m = jnp.where(m >= _DIM, m - _DIM, m)
                    dtv = lax.shift_right_logical(m, 3)
                    inner = lax.shift_left(lax.bitwise_and(m, 7), 7)
                    vals = [plsc.load_gather(src, [rowbase[g], m])
                            for g in range(8)]
                    for g in range(8):
                        plsc.store_scatter(dst, [dtv, inner + rowbase[g]],
                                           vals[g])
                return carry

            lax.fori_loop(0, _DIM // 8, d0_body, 0)

    def fire_writes(p, i):
        l0, bt = coords(i)
        pltpu.async_copy(
            tiles_v[p],
            out_hbm.at[pl.ds(l0, _SL), pl.ds(0, _DT),
                       pl.ds(bt * 1024, 1024)],
            semw[p])

    def drain_writes(p):
        pltpu.make_async_copy(
            tiles_v[p],
            out_hbm.at[pl.ds(0, _SL), pl.ds(0, _DT), pl.ds(0, 1024)],
            semw[p]).wait()

    fire_idx(0, 0)
    drain_idx(0)
    scale_idx(0)
    fire_gather(0)
    fire_idx(1, 1)

    def step(s, carry):
        for p in (0, 1):
            i = 2 * s + p

            @pl.when(s >= 1)
            def _(p=p):
                drain_writes(p)

            drain_gather(p)

            @pl.when(i + 1 < _NBLK)
            def _(p=p):
                drain_idx(1 - p)
                scale_idx(1 - p)
                fire_gather(1 - p)

            @pl.when(i + 2 < _NBLK)
            def _(p=p, i=i):
                fire_idx(p, i + 2)

            swizzle(p)
            fire_writes(p, i)

        return carry

    lax.fori_loop(0, _NBLK // 2, step, 0)
    drain_writes(0)
    drain_writes(1)


@jax.jit
def _embedding_lookup(xt, table):
    mesh = plsc.VectorSubcoreMesh(core_axis_name="c", subcore_axis_name="s")
    return pl.kernel(
        _gather_body,
        mesh=mesh,
        out_type=jax.ShapeDtypeStruct((_L, _DT, _BT * 1024), jnp.float32),
        scratch_types=[
            pltpu.VMEM((_SL, 128), jnp.int32),
            pltpu.VMEM((_SL, 128), jnp.int32),
            pltpu.VMEM((_SL, 128), jnp.int32),
            pltpu.VMEM((_SL, 128), jnp.int32),
            pltpu.VMEM((_SL, 128, _DIM), jnp.float32),
            pltpu.VMEM((_SL, 128, _DIM), jnp.float32),
            pltpu.VMEM((_SL, _DT, 1024), jnp.float32),
            pltpu.VMEM((_SL, _DT, 1024), jnp.float32),
            pltpu.SemaphoreType.DMA,
            pltpu.SemaphoreType.DMA,
            pltpu.SemaphoreType.DMA,
            pltpu.SemaphoreType.DMA,
            pltpu.SemaphoreType.DMA,
            pltpu.SemaphoreType.DMA,
        ],
        compiler_params=pltpu.CompilerParams(use_tc_tiling_on_sc=False,
                                             needs_layout_passes=False),
    )(xt, table)


def kernel(x, table):
    xt = x.T.astype(jnp.int32)
    tp = jnp.pad(table, ((0, 0), (0, 96))).reshape(4 * 1000000, 32)
    q = _embedding_lookup(xt, tp)
    q = q.reshape(_L, _DT, _BT, 8, 128)
    return q.transpose(2, 4, 0, 1, 3).reshape(_B, _L, _DIM)
